# Initial kernel scaffold; baseline (speedup 1.0000x reference)
import jax
import jax.numpy as jnp
from jax.experimental import pallas as pl


def kernel(W_emb, leavesList, ancestorsList, W_attention, b_attention, v_attention):
    raise NotImplementedError("write your pallas kernel here")


# speedup vs baseline: 4.4682x; 4.4682x over previous
import functools

import jax
import jax.numpy as jnp
from jax import lax
from jax.experimental import pallas as pl
from jax.experimental.pallas import tpu as pltpu
from jax.experimental.pallas import tpu_sc as plsc

_VOCAB = 100000
_D = 128
_ATT = 128
_G, _N, _A = 4, 2048, 64
_NC, _NS = 2, 16
_NW = _NC * _NS

_VB = 4000


def _score_body(w_ref, wa_ref, b_ref, v_ref, o_ref):
    x = w_ref[...]
    h = jnp.tanh(
        jnp.dot(x, wa_ref[...], preferred_element_type=jnp.float32)
        + b_ref[...][None, :]
    )
    o_ref[...] = jnp.dot(h, v_ref[...],
                         preferred_element_type=jnp.float32).reshape(1, 1, _VB)


def _scores(W_emb, W_att, b_att, v_att):
    nb = _VOCAB // _VB
    out = pl.pallas_call(
        _score_body,
        grid=(nb,),
        in_specs=[
            pl.BlockSpec((_VB, _D), lambda i: (i, 0)),
            pl.BlockSpec((_D, _ATT), lambda i: (0, 0)),
            pl.BlockSpec((_ATT,), lambda i: (0,)),
            pl.BlockSpec((_ATT,), lambda i: (0,)),
        ],
        out_specs=pl.BlockSpec((1, 1, _VB), lambda i: (i, 0, 0)),
        out_shape=jax.ShapeDtypeStruct((nb, 1, _VB), jnp.float32),
    )(W_emb, W_att, b_att, v_att)
    return out.reshape(_VOCAB)


_BCH = 4096
_PER_W = (_G * _N * _D) // _NW


def _gather_scores(idx_all, s):
    mesh = plsc.VectorSubcoreMesh(core_axis_name="c", subcore_axis_name="s")

    @functools.partial(
        pl.kernel, mesh=mesh,
        compiler_params=pltpu.CompilerParams(needs_layout_passes=False),
        out_type=jax.ShapeDtypeStruct((_G * _N * _D,), jnp.float32),
        scratch_types=[
            pltpu.VMEM((_VOCAB,), jnp.float32),
            pltpu.VMEM((_BCH,), jnp.int32),
            pltpu.VMEM((_BCH,), jnp.float32),
        ],
    )
    def k(idx_hbm, s_hbm, out_hbm, s_v, idx_v, out_v):
        wid = lax.axis_index("s") * _NC + lax.axis_index("c")
        pltpu.sync_copy(s_hbm, s_v)
        base = wid * _PER_W
        for ch in range(_PER_W // _BCH):
            off = base + ch * _BCH
            pltpu.sync_copy(idx_hbm.at[pl.ds(off, _BCH)], idx_v)

            def body(i, carry):
                iv = idx_v[pl.ds(i * 16, 16)]
                out_v[pl.ds(i * 16, 16)] = plsc.load_gather(s_v, [iv])
                return carry

            lax.fori_loop(0, _BCH // 16, body, 0, unroll=8)
            pltpu.sync_copy(out_v, out_hbm.at[pl.ds(off, _BCH)])

    return k(idx_all, s)


def _softmax_body(x_ref, o_ref):
    x = x_ref[...]
    m = jnp.max(x, axis=1, keepdims=True)
    e = jnp.exp(x - m)
    o_ref[...] = e / jnp.sum(e, axis=1, keepdims=True)


def _softmax_n(pre):
    return pl.pallas_call(
        _softmax_body,
        grid=(_G,),
        in_specs=[pl.BlockSpec((1, _N, _D), lambda g: (g, 0, 0))],
        out_specs=pl.BlockSpec((1, _N, _D), lambda g: (g, 0, 0)),
        out_shape=jax.ShapeDtypeStruct((_G, _N, _D), jnp.float32),
    )(pre)


_NCHUNKS = 64


def _weighted_sum(W_emb, anc_r, att):
    mesh = plsc.VectorSubcoreMesh(core_axis_name="c", subcore_axis_name="s")

    @functools.partial(
        pl.kernel, mesh=mesh,
        compiler_params=pltpu.CompilerParams(needs_layout_passes=False),
        out_type=jax.ShapeDtypeStruct((_G * _A, _D), jnp.float32),
        scratch_types=[
            pltpu.VMEM((_NCHUNKS, 128), jnp.int32),
            pltpu.VMEM((2, 128, _D), jnp.float32),
            pltpu.VMEM((128, _D), jnp.float32),
            pltpu.VMEM((_A, _D), jnp.float32),
            pltpu.VMEM((_A,), jnp.int32),
            pltpu.VMEM_SHARED((2 * _A, _D), jnp.float32),
            pltpu.SemaphoreType.DMA,
            pltpu.SemaphoreType.DMA,
        ],
    )
    def k(emb_hbm, anc_hbm, att_hbm, out_hbm,
          idx_v, rows_v, att_v, acc_v, iota_v, shared_acc, sem0, sem1):
        c = lax.axis_index("c")
        s = lax.axis_index("s")
        sems = (sem0, sem1)
        zeros = jnp.zeros((16,), jnp.float32)

        def zero_acc():
            def zbody(a, carry):
                for ec in range(8):
                    acc_v[a, pl.ds(ec * 16, 16)] = zeros
                return carry
            lax.fori_loop(0, _A, zbody, 0, unroll=8)

        zero_acc()

        @pl.when(s == 0)
        def _():
            pltpu.sync_copy(acc_v, shared_acc.at[pl.ds(0, _A)])
            pltpu.sync_copy(acc_v, shared_acc.at[pl.ds(_A, _A)])

        plsc.subcore_barrier()

        def group_body(gl, carry):
            g = 2 * c + gl
            for kb in range(_A // 16):
                iota_v[pl.ds(kb * 16, 16)] = (
                    lax.iota(jnp.int32, 16) + (kb * 16) + gl * _A)
            n0 = s * 128
            pltpu.sync_copy(att_hbm.at[g, pl.ds(n0, 128)], att_v)
            pltpu.sync_copy(anc_hbm.at[g, s], idx_v)
            pltpu.make_async_copy(
                emb_hbm.at[idx_v.at[0]], rows_v.at[0], sems[0]).start()

            def chunk_pair(jj, carry2):
                for b in range(2):
                    j = 2 * jj + b

                    @pl.when(j + 1 < _NCHUNKS)
                    def _():
                        pltpu.make_async_copy(
                            emb_hbm.at[idx_v.at[j + 1]],
                            rows_v.at[1 - b], sems[1 - b]).start()

                    pltpu.make_async_copy(
                        emb_hbm.at[idx_v.at[j]], rows_v.at[b], sems[b]).wait()

                    for nn in range(2):
                        n_l = 2 * j + nn
                        for ec in range(8):
                            av = att_v[n_l, pl.ds(ec * 16, 16)]

                            def abody(a, carry3):
                                r = rows_v[b, nn * 64 + a, pl.ds(ec * 16, 16)]
                                plsc.addupdate(
                                    acc_v.at[a, pl.ds(ec * 16, 16)], r * av)
                                return carry3

                            lax.fori_loop(0, _A, abody, 0, unroll=16)
                return carry2

            lax.fori_loop(0, _NCHUNKS // 2, chunk_pair, 0)
            pltpu.sync_copy(acc_v, shared_acc.at[iota_v], add=True)
            zero_acc()
            return carry

        lax.fori_loop(0, 2, group_body, 0)
        plsc.subcore_barrier()

        @pl.when(s == 0)
        def _():
            pltpu.sync_copy(shared_acc, out_hbm.at[pl.ds(c * 2 * _A, 2 * _A)])

    return k(W_emb, anc_r, att)


def kernel(W_emb, leavesList, ancestorsList, W_attention, b_attention,
           v_attention):
    s = _scores(W_emb, W_attention, b_attention, v_attention)
    idx_all = jnp.concatenate([leavesList, ancestorsList], axis=2).reshape(-1)
    pre = _gather_scores(idx_all, s).reshape(_G, _N, _D)
    att = _softmax_n(pre)
    anc_r = ancestorsList.reshape(_G, _NS, _NCHUNKS, 128)
    out = _weighted_sum(W_emb, anc_r, att).reshape(_G, _A, _D)
    return tuple(out[g:g + 1] for g in range(_G))


# speedup vs baseline: 10.2378x; 2.2913x over previous
import functools

import jax
import jax.numpy as jnp
from jax import lax
from jax.experimental import pallas as pl
from jax.experimental.pallas import tpu as pltpu
from jax.experimental.pallas import tpu_sc as plsc

_VOCAB = 100000
_D = 128
_ATT = 128
_G, _N, _A = 4, 2048, 64
_NC, _NS = 2, 16
_NW = _NC * _NS

_VB = 4000


def _score_body(w_ref, wa_ref, b_ref, v_ref, o_ref):
    x = w_ref[...]
    h = jnp.tanh(
        jnp.dot(x, wa_ref[...], preferred_element_type=jnp.float32)
        + b_ref[...][None, :]
    )
    o_ref[...] = jnp.dot(h, v_ref[...],
                         preferred_element_type=jnp.float32).reshape(1, 1, _VB)


def _scores(W_emb, W_att, b_att, v_att):
    nb = _VOCAB // _VB
    out = pl.pallas_call(
        _score_body,
        grid=(nb,),
        in_specs=[
            pl.BlockSpec((_VB, _D), lambda i: (i, 0)),
            pl.BlockSpec((_D, _ATT), lambda i: (0, 0)),
            pl.BlockSpec((_ATT,), lambda i: (0,)),
            pl.BlockSpec((_ATT,), lambda i: (0,)),
        ],
        out_specs=pl.BlockSpec((1, 1, _VB), lambda i: (i, 0, 0)),
        out_shape=jax.ShapeDtypeStruct((nb, 1, _VB), jnp.float32),
    )(W_emb, W_att, b_att, v_att)
    return out.reshape(_VOCAB)


_BCH = 4096
_PER_W = (_G * _N * _D) // _NW


def _gather_scores(idx_all, s):
    mesh = plsc.VectorSubcoreMesh(core_axis_name="c", subcore_axis_name="s")

    @functools.partial(
        pl.kernel, mesh=mesh,
        compiler_params=pltpu.CompilerParams(needs_layout_passes=False),
        out_type=jax.ShapeDtypeStruct((_G * _N * _D,), jnp.float32),
        scratch_types=[
            pltpu.VMEM((_VOCAB,), jnp.float32),
            pltpu.VMEM((_BCH,), jnp.int32),
            pltpu.VMEM((_BCH,), jnp.float32),
        ],
    )
    def k(idx_hbm, s_hbm, out_hbm, s_v, idx_v, out_v):
        wid = lax.axis_index("s") * _NC + lax.axis_index("c")
        pltpu.sync_copy(s_hbm, s_v)
        base = wid * _PER_W
        for ch in range(_PER_W // _BCH):
            off = base + ch * _BCH
            pltpu.sync_copy(idx_hbm.at[pl.ds(off, _BCH)], idx_v)

            @plsc.parallel_loop(0, _BCH // 16, unroll=8)
            def body(i):
                iv = idx_v[pl.ds(i * 16, 16)]
                out_v[pl.ds(i * 16, 16)] = plsc.load_gather(s_v, [iv])

            pltpu.sync_copy(out_v, out_hbm.at[pl.ds(off, _BCH)])

    return k(idx_all, s)


def _softmax_body(x_ref, o_ref):
    x = x_ref[...]
    m = jnp.max(x, axis=1, keepdims=True)
    e = jnp.exp(x - m)
    o_ref[...] = e / jnp.sum(e, axis=1, keepdims=True)


def _softmax_n(pre):
    return pl.pallas_call(
        _softmax_body,
        grid=(_G,),
        in_specs=[pl.BlockSpec((1, _N, _D), lambda g: (g, 0, 0))],
        out_specs=pl.BlockSpec((1, _N, _D), lambda g: (g, 0, 0)),
        out_shape=jax.ShapeDtypeStruct((_G, _N, _D), jnp.float32),
    )(pre)


_CN = 4
_NCHUNKS = 128 // _CN


def _weighted_sum(W_emb, anc_r, att):
    mesh = plsc.VectorSubcoreMesh(core_axis_name="c", subcore_axis_name="s")

    @functools.partial(
        pl.kernel, mesh=mesh,
        compiler_params=pltpu.CompilerParams(needs_layout_passes=False),
        out_type=jax.ShapeDtypeStruct((_G * _A, _D), jnp.float32),
        scratch_types=[
            pltpu.VMEM((64, 128), jnp.int32),
            pltpu.VMEM((2, 2, 128, _D), jnp.float32),
            pltpu.VMEM((128, _D), jnp.float32),
            pltpu.VMEM((_A, _D), jnp.float32),
            pltpu.VMEM((_A,), jnp.int32),
            pltpu.VMEM_SHARED((2 * _A, _D), jnp.float32),
            pltpu.SemaphoreType.DMA,
            pltpu.SemaphoreType.DMA,
        ],
    )
    def k(emb_hbm, anc_hbm, att_hbm, out_hbm,
          idx_v, rows_v, att_v, acc_v, iota_v, shared_acc, sem0, sem1):
        c = lax.axis_index("c")
        s = lax.axis_index("s")
        sems = (sem0, sem1)
        zeros = jnp.zeros((16,), jnp.float32)

        def zero_acc():
            @plsc.parallel_loop(0, _A, unroll=8)
            def zbody(a):
                for ec in range(8):
                    acc_v[a, pl.ds(ec * 16, 16)] = zeros

        def start_chunk(j, b):
            pltpu.make_async_copy(
                emb_hbm.at[idx_v.at[2 * j]], rows_v.at[b, 0], sems[b]).start()
            pltpu.make_async_copy(
                emb_hbm.at[idx_v.at[2 * j + 1]], rows_v.at[b, 1],
                sems[b]).start()

        def wait_chunk(j, b):
            pltpu.make_async_copy(
                emb_hbm.at[idx_v.at[2 * j]], rows_v.at[b, 0], sems[b]).wait()
            pltpu.make_async_copy(
                emb_hbm.at[idx_v.at[2 * j + 1]], rows_v.at[b, 1],
                sems[b]).wait()

        zero_acc()

        @pl.when(s == 0)
        def _():
            pltpu.sync_copy(acc_v, shared_acc.at[pl.ds(0, _A)])
            pltpu.sync_copy(acc_v, shared_acc.at[pl.ds(_A, _A)])

        plsc.subcore_barrier()

        def group_body(gl, carry):
            g = 2 * c + gl
            for kb in range(_A // 16):
                iota_v[pl.ds(kb * 16, 16)] = (
                    lax.iota(jnp.int32, 16) + (kb * 16) + gl * _A)
            n0 = s * 128
            pltpu.sync_copy(att_hbm.at[g, pl.ds(n0, 128)], att_v)
            pltpu.sync_copy(anc_hbm.at[g, s], idx_v)
            start_chunk(0, 0)

            def chunk_pair(jj, carry2):
                for b in range(2):
                    j = 2 * jj + b

                    @pl.when(j + 1 < _NCHUNKS)
                    def _():
                        start_chunk(j + 1, 1 - b)

                    wait_chunk(j, b)

                    avs = [
                        [att_v[_CN * j + nn, pl.ds(ec * 16, 16)]
                         for ec in range(8)]
                        for nn in range(_CN)
                    ]

                    def abody(a, carry3):
                        for ec in range(8):
                            sl = pl.ds(ec * 16, 16)
                            t = rows_v[b, 0, a, sl] * avs[0][ec]
                            t = t + rows_v[b, 0, 64 + a, sl] * avs[1][ec]
                            t = t + rows_v[b, 1, a, sl] * avs[2][ec]
                            t = t + rows_v[b, 1, 64 + a, sl] * avs[3][ec]
                            plsc.addupdate(acc_v.at[a, sl], t)
                        return carry3

                    lax.fori_loop(0, _A, abody, 0, unroll=2)
                return carry2

            lax.fori_loop(0, _NCHUNKS // 2, chunk_pair, 0)
            pltpu.sync_copy(acc_v, shared_acc.at[iota_v], add=True)
            zero_acc()
            return carry

        lax.fori_loop(0, 2, group_body, 0)
        plsc.subcore_barrier()

        @pl.when(s == 0)
        def _():
            pltpu.sync_copy(shared_acc, out_hbm.at[pl.ds(c * 2 * _A, 2 * _A)])

    return k(W_emb, anc_r, att)


def kernel(W_emb, leavesList, ancestorsList, W_attention, b_attention,
           v_attention):
    s = _scores(W_emb, W_attention, b_attention, v_attention)
    idx_all = jnp.concatenate([leavesList, ancestorsList], axis=2).reshape(-1)
    pre = _gather_scores(idx_all, s).reshape(_G, _N, _D)
    att = _softmax_n(pre)
    anc_r = ancestorsList.reshape(_G, _NS, 64, 128)
    out = _weighted_sum(W_emb, anc_r, att).reshape(_G, _A, _D)
    return tuple(out[g:g + 1] for g in range(_G))
